# baseline (speedup 1.0000x reference)
import functools

import jax
import jax.numpy as jnp
from jax import lax
from jax.experimental import pallas as pl
from jax.experimental.pallas import tpu as pltpu
from jax.experimental.pallas import tpu_sc as plsc

FD = 10000
BATCH = 1024
SCALE = 1.2

NC, NS, L = 2, 16, 16
NW = NC * NS
RPW = BATCH // NW


def _copy_body(x_ref, o_ref):
    o_ref[...] = x_ref[...]


def _logits_copy_t(xt):
    RB = 1000
    return pl.pallas_call(
        _copy_body,
        grid=(FD // RB,),
        in_specs=[pl.BlockSpec((RB, BATCH), lambda i: (i, 0))],
        out_specs=pl.BlockSpec((RB, BATCH), lambda i: (i, 0)),
        out_shape=jax.ShapeDtypeStruct((FD, BATCH), jnp.float32),
        compiler_params=pltpu.CompilerParams(
            dimension_semantics=("parallel",)),
    )(xt)


def _sc_routed_build():
    mesh = plsc.VectorSubcoreMesh(core_axis_name="c", subcore_axis_name="s")

    o2 = jax.ShapeDtypeStruct((2, BATCH), jnp.float32)

    @functools.partial(
        pl.kernel,
        out_type=(o2, o2, o2, o2),
        mesh=mesh,
        scratch_types=(
            pltpu.VMEM((RPW,), jnp.int32),
            pltpu.VMEM((RPW,), jnp.int32),
            pltpu.VMEM((RPW,), jnp.int32),
            pltpu.VMEM((RPW, 128), jnp.float32),
            pltpu.VMEM((RPW, 128), jnp.float32),
            pltpu.VMEM((RPW, 128), jnp.float32),
            pltpu.VMEM((RPW, 128), jnp.float32),
            pltpu.VMEM((8, RPW), jnp.float32),
            pltpu.SemaphoreType.DMA,
            pltpu.SemaphoreType.DMA,
            pltpu.SemaphoreType.DMA,
            pltpu.SemaphoreType.DMA,
        ),
    )
    def k(xt, labels, ccp, csp,
          gps_o, size_o, center_o, reg_o,
          lab_v, ilat_v, ilon_v, glat_v, glon_v, gc_v, gs_v, st_v,
          sem0, sem1, sem2, sem3):
        wid = lax.axis_index("s") * NC + lax.axis_index("c")
        base = wid * RPW
        pltpu.sync_copy(labels.at[pl.ds(base, RPW)], lab_v)

        iota = lax.iota(jnp.int32, L)
        zero16 = iota.astype(jnp.float32) * 0.0
        for j in range(RPW // L):
            lab16 = lab_v[pl.ds(j * L, L)]
            el2 = lab16 * 2
            ilat_v[pl.ds(j * L, L)] = FD + el2
            ilon_v[pl.ds(j * L, L)] = FD + el2 + 1

        win = lax.shift_right_logical(base, 7) * 128
        cp0 = pltpu.async_copy(xt.at[ilat_v, pl.ds(win, 128)], glat_v, sem0)
        cp1 = pltpu.async_copy(xt.at[ilon_v, pl.ds(win, 128)], glon_v, sem1)
        cp2 = pltpu.async_copy(ccp.at[lab_v], gc_v, sem2)
        cp3 = pltpu.async_copy(csp.at[lab_v], gs_v, sem3)
        cp0.wait()
        cp1.wait()
        cp2.wait()
        cp3.wait()

        woff = jnp.bitwise_and(base, 127)
        for j in range(RPW // L):
            rlat = zero16
            rlon = zero16
            clat = zero16
            clon = zero16
            slat = zero16
            slon = zero16
            for i in range(L):
                r = j * L + i
                w = woff + (r // L) * L
                pick = iota == i
                rlat = jnp.where(pick, glat_v[r, pl.ds(w, L)][r % L], rlat)
                rlon = jnp.where(pick, glon_v[r, pl.ds(w, L)][r % L], rlon)
                vc = gc_v[r, pl.ds(0, L)]
                clat = jnp.where(pick, vc[0], clat)
                clon = jnp.where(pick, vc[1], clon)
                vs = gs_v[r, pl.ds(0, L)]
                slat = jnp.where(pick, vs[0], slat)
                slon = jnp.where(pick, vs[1], slon)

            elat = jnp.exp(rlat * 2.0)
            rlat = SCALE * (1.0 - 2.0 / (elat + 1.0))
            elon = jnp.exp(rlon * 2.0)
            rlon = SCALE * (1.0 - 2.0 / (elon + 1.0))
            glat = jnp.clip(clat + rlat * slat * 0.5, -1.0, 1.0) * 90.0
            glon = jnp.clip(clon + rlon * slon * 0.5, -1.0, 1.0) * 180.0

            sl = pl.ds(j * L, L)
            st_v[0, sl] = glat
            st_v[1, sl] = glon
            st_v[2, sl] = 2.0 / slat
            st_v[3, sl] = 2.0 / slon
            st_v[4, sl] = clat
            st_v[5, sl] = clon
            st_v[6, sl] = rlat
            st_v[7, sl] = rlon

        bsl = pl.ds(base, RPW)
        pltpu.sync_copy(st_v.at[0], gps_o.at[0, bsl])
        pltpu.sync_copy(st_v.at[1], gps_o.at[1, bsl])
        pltpu.sync_copy(st_v.at[2], size_o.at[0, bsl])
        pltpu.sync_copy(st_v.at[3], size_o.at[1, bsl])
        pltpu.sync_copy(st_v.at[4], center_o.at[0, bsl])
        pltpu.sync_copy(st_v.at[5], center_o.at[1, bsl])
        pltpu.sync_copy(st_v.at[6], reg_o.at[0, bsl])
        pltpu.sync_copy(st_v.at[7], reg_o.at[1, bsl])

    return k


def kernel(x, gt_label, cell_center, cell_size):
    xt = x.T
    logits = _logits_copy_t(xt).T

    ccp = jnp.pad(cell_center, ((0, 0), (0, 126)))
    csp = jnp.pad(cell_size, ((0, 0), (0, 126)))

    sc_k = _sc_routed_build()
    gps_t, size_t, center_t, reg_t = sc_k(
        xt, gt_label.astype(jnp.int32), ccp, csp)

    return (logits, gps_t.T, size_t.T, center_t.T, reg_t.T)


# speedup vs baseline: 1.0260x; 1.0260x over previous
import functools

import jax
import jax.numpy as jnp
from jax import lax
from jax.experimental import pallas as pl
from jax.experimental.pallas import tpu as pltpu
from jax.experimental.pallas import tpu_sc as plsc

FD = 10000
BATCH = 1024
SCALE = 1.2

NC, NS, L = 2, 16, 16
NW = NC * NS
RPW = BATCH // NW


def _copy_body(x_ref, o_ref):
    o_ref[...] = x_ref[...]


def _logits_copy_t(xt):
    RB = 2000
    return pl.pallas_call(
        _copy_body,
        grid=(FD // RB,),
        in_specs=[pl.BlockSpec((RB, BATCH), lambda i: (i, 0))],
        out_specs=pl.BlockSpec((RB, BATCH), lambda i: (i, 0)),
        out_shape=jax.ShapeDtypeStruct((FD, BATCH), jnp.float32),
        compiler_params=pltpu.CompilerParams(
            dimension_semantics=("parallel",)),
    )(xt)


def _sc_routed_build():
    mesh = plsc.VectorSubcoreMesh(core_axis_name="c", subcore_axis_name="s")

    o2 = jax.ShapeDtypeStruct((2, BATCH), jnp.float32)

    @functools.partial(
        pl.kernel,
        out_type=(o2, o2, o2, o2),
        mesh=mesh,
        scratch_types=(
            pltpu.VMEM((RPW,), jnp.int32),
            pltpu.VMEM((RPW,), jnp.int32),
            pltpu.VMEM((RPW,), jnp.int32),
            pltpu.VMEM((RPW, 128), jnp.float32),
            pltpu.VMEM((RPW, 128), jnp.float32),
            pltpu.VMEM((RPW, 128), jnp.float32),
            pltpu.VMEM((RPW, 128), jnp.float32),
            pltpu.VMEM((8, RPW), jnp.float32),
            pltpu.SemaphoreType.DMA,
            pltpu.SemaphoreType.DMA,
            pltpu.SemaphoreType.DMA,
            pltpu.SemaphoreType.DMA,
        ),
    )
    def k(xt, labels, ccp, csp,
          gps_o, size_o, center_o, reg_o,
          lab_v, ilat_v, ilon_v, glat_v, glon_v, gc_v, gs_v, st_v,
          sem0, sem1, sem2, sem3):
        wid = lax.axis_index("s") * NC + lax.axis_index("c")
        base = wid * RPW
        pltpu.sync_copy(labels.at[pl.ds(base, RPW)], lab_v)

        iota = lax.iota(jnp.int32, L)
        zero16 = iota.astype(jnp.float32) * 0.0
        for j in range(RPW // L):
            lab16 = lab_v[pl.ds(j * L, L)]
            el2 = lab16 * 2
            ilat_v[pl.ds(j * L, L)] = FD + el2
            ilon_v[pl.ds(j * L, L)] = FD + el2 + 1

        win = lax.shift_right_logical(base, 7) * 128
        cp0 = pltpu.async_copy(xt.at[ilat_v, pl.ds(win, 128)], glat_v, sem0)
        cp1 = pltpu.async_copy(xt.at[ilon_v, pl.ds(win, 128)], glon_v, sem1)
        cp2 = pltpu.async_copy(ccp.at[lab_v], gc_v, sem2)
        cp3 = pltpu.async_copy(csp.at[lab_v], gs_v, sem3)
        cp0.wait()
        cp1.wait()
        cp2.wait()
        cp3.wait()

        woff = jnp.bitwise_and(base, 127)
        for j in range(RPW // L):
            rlat = zero16
            rlon = zero16
            clat = zero16
            clon = zero16
            slat = zero16
            slon = zero16
            for i in range(L):
                r = j * L + i
                w = woff + (r // L) * L
                pick = iota == i
                rlat = jnp.where(pick, glat_v[r, pl.ds(w, L)][r % L], rlat)
                rlon = jnp.where(pick, glon_v[r, pl.ds(w, L)][r % L], rlon)
                vc = gc_v[r, pl.ds(0, L)]
                clat = jnp.where(pick, vc[0], clat)
                clon = jnp.where(pick, vc[1], clon)
                vs = gs_v[r, pl.ds(0, L)]
                slat = jnp.where(pick, vs[0], slat)
                slon = jnp.where(pick, vs[1], slon)

            elat = jnp.exp(rlat * 2.0)
            rlat = SCALE * (1.0 - 2.0 / (elat + 1.0))
            elon = jnp.exp(rlon * 2.0)
            rlon = SCALE * (1.0 - 2.0 / (elon + 1.0))
            glat = jnp.clip(clat + rlat * slat * 0.5, -1.0, 1.0) * 90.0
            glon = jnp.clip(clon + rlon * slon * 0.5, -1.0, 1.0) * 180.0

            sl = pl.ds(j * L, L)
            st_v[0, sl] = glat
            st_v[1, sl] = glon
            st_v[2, sl] = 2.0 / slat
            st_v[3, sl] = 2.0 / slon
            st_v[4, sl] = clat
            st_v[5, sl] = clon
            st_v[6, sl] = rlat
            st_v[7, sl] = rlon

        bsl = pl.ds(base, RPW)
        pltpu.sync_copy(st_v.at[0], gps_o.at[0, bsl])
        pltpu.sync_copy(st_v.at[1], gps_o.at[1, bsl])
        pltpu.sync_copy(st_v.at[2], size_o.at[0, bsl])
        pltpu.sync_copy(st_v.at[3], size_o.at[1, bsl])
        pltpu.sync_copy(st_v.at[4], center_o.at[0, bsl])
        pltpu.sync_copy(st_v.at[5], center_o.at[1, bsl])
        pltpu.sync_copy(st_v.at[6], reg_o.at[0, bsl])
        pltpu.sync_copy(st_v.at[7], reg_o.at[1, bsl])

    return k


def kernel(x, gt_label, cell_center, cell_size):
    xt = x.T
    logits = _logits_copy_t(xt).T

    ccp = jnp.pad(cell_center, ((0, 0), (0, 126)))
    csp = jnp.pad(cell_size, ((0, 0), (0, 126)))

    sc_k = _sc_routed_build()
    gps_t, size_t, center_t, reg_t = sc_k(
        xt, gt_label.astype(jnp.int32), ccp, csp)

    return (logits, gps_t.T, size_t.T, center_t.T, reg_t.T)
